# baseline (speedup 1.0000x reference)
import jax
import jax.numpy as jnp
from jax import lax
from jax.experimental import pallas as pl
from jax.experimental.pallas import tpu as pltpu

B, C, H, W = 8, 96, 224, 224
R = B * C


def _tc_body(idx_ref, x_ref, o_ref):
    o_ref[...] = x_ref[...]


@jax.jit
def kernel(x, perm):
    x3 = x.reshape(R, H, W)
    rows = jnp.arange(R, dtype=jnp.int32)
    src = (rows // C) * C + perm.astype(jnp.int32)[rows % C]

    grid_spec = pltpu.PrefetchScalarGridSpec(
        num_scalar_prefetch=1,
        grid=(R,),
        in_specs=[pl.BlockSpec((1, H, W), lambda i, idx: (idx[i], 0, 0))],
        out_specs=pl.BlockSpec((1, H, W), lambda i, idx: (i, 0, 0)),
    )
    out3 = pl.pallas_call(
        _tc_body,
        grid_spec=grid_spec,
        out_shape=jax.ShapeDtypeStruct((R, H, W), jnp.float32),
    )(src, x3)
    return out3.reshape(B, C, H, W)


# speedup vs baseline: 1.7535x; 1.7535x over previous
import jax
import jax.numpy as jnp
import numpy as np
from jax import lax
from jax.experimental import pallas as pl
from jax.experimental.pallas import tpu as pltpu
from jax.experimental.pallas import tpu_sc as plsc

B, C, H, W = 8, 96, 224, 224
R = B * C
BT = 6
RSC = BT * C
NC, NS = 2, 16
NW = NC * NS
M = RSC // NW
IDXW = 32

_out_plane = np.zeros((NW, IDXW), dtype=np.int32)
for _w in range(NW):
    _out_plane[_w, :M] = _w * M + np.arange(M)


def _sc_body(x_hbm, idx_hbm, out_hbm, idx_v, bufs, gsems, ssems):
    c = lax.axis_index("c")
    s = lax.axis_index("s")
    w = s * NC + c
    base = w * M
    pltpu.sync_copy(idx_hbm.at[w], idx_v)

    lanes = lax.broadcasted_iota(jnp.int32, (16,), 0)
    vecs = [idx_v[pl.ds(16 * g, 16)] for g in range(IDXW // 16)]

    def src_of(j):
        vec, lane = vecs[j // 16], j % 16
        return lax.reduce_max(jnp.where(lanes == lane, vec, 0), (0,))

    def gather(j):
        b = j % 2
        return pltpu.async_copy(x_hbm.at[pl.ds(src_of(j), 1)], bufs[b],
                                gsems[b])

    def wait_gather(j):
        b = j % 2
        pltpu.make_async_copy(x_hbm.at[pl.ds(src_of(j), 1)], bufs[b],
                              gsems[b]).wait()

    def scatter(j):
        b = j % 2
        return pltpu.async_copy(bufs[b], out_hbm.at[pl.ds(base + j, 1)],
                                ssems[b])

    def wait_scatter(j):
        b = j % 2
        pltpu.make_async_copy(bufs[b], out_hbm.at[pl.ds(base + j, 1)],
                              ssems[b]).wait()

    gather(0)
    gather(1)
    for j in range(M):
        wait_gather(j)
        scatter(j)
        if j + 2 < M:
            wait_scatter(j)
            gather(j + 2)
    wait_scatter(M - 2)
    wait_scatter(M - 1)


def _tc_body(idx_ref, x_ref, o_ref):
    o_ref[...] = x_ref[...]


@jax.jit
def kernel(x, perm):
    x3 = x.reshape(R, H, W)
    rows = jnp.arange(RSC, dtype=jnp.int32)
    src = (rows // C) * C + perm.astype(jnp.int32)[rows % C]
    idx_rows = src[_out_plane]

    mesh = plsc.VectorSubcoreMesh(core_axis_name="c", subcore_axis_name="s")
    sc_out = pl.kernel(
        _sc_body,
        out_type=jax.ShapeDtypeStruct((RSC, H, W), jnp.float32),
        mesh=mesh,
        compiler_params=pltpu.CompilerParams(use_tc_tiling_on_sc=True,
                                             needs_layout_passes=False),
        scratch_types=[
            pltpu.VMEM((IDXW,), jnp.int32),
            [pltpu.VMEM((1, H, W), jnp.float32) for _ in range(2)],
            [pltpu.SemaphoreType.DMA for _ in range(2)],
            [pltpu.SemaphoreType.DMA for _ in range(2)],
        ],
    )(x3, idx_rows)

    grid_spec = pltpu.PrefetchScalarGridSpec(
        num_scalar_prefetch=1,
        grid=(C,),
        in_specs=[pl.BlockSpec((B - BT, 1, H, W),
                               lambda i, idx: (BT // (B - BT), idx[i], 0, 0))],
        out_specs=pl.BlockSpec((B - BT, 1, H, W),
                               lambda i, idx: (0, i, 0, 0)),
    )
    tc_out = pl.pallas_call(
        _tc_body,
        grid_spec=grid_spec,
        out_shape=jax.ShapeDtypeStruct((B - BT, C, H, W), jnp.float32),
    )(perm.astype(jnp.int32), x)

    out3 = jnp.concatenate([sc_out, tc_out.reshape((B - BT) * C, H, W)],
                           axis=0)
    return out3.reshape(B, C, H, W)
